# baseline (device time: 1002958 ns/iter reference)
import jax
import jax.numpy as jnp
from jax import lax
from jax.experimental import pallas as pl
from jax.experimental.pallas import tpu as pltpu

N_CHUNKS = 16
N_SLOTS = 4


def kernel(x):
    m_per, n = x.shape
    m_glob = 2 * m_per
    m_half = m_per // 2
    ck = m_half // N_CHUNKS
    ckv = m_per // N_CHUNKS

    def body(x_ref, out_ref, s1_send, s1_recv, s2_send, s2_recv,
             vbuf, h2v_sem, v2h_sem):
        my_x = lax.axis_index("x")
        my_y = lax.axis_index("y")
        y_peer = (my_x, 1 - my_y)
        x_peer = (1 - my_x, my_y)

        barrier_sem = pltpu.get_barrier_semaphore()
        for peer in (y_peer, x_peer):
            pl.semaphore_signal(
                barrier_sem, inc=1, device_id=peer,
                device_id_type=pl.DeviceIdType.MESH,
            )
        pl.semaphore_wait(barrier_sem, 2)

        my_half_src = my_x * m_half
        s1_rows = (1 - my_y) * m_per + my_x * m_half
        s2_rows = (1 - my_y) * m_per + (1 - my_x) * m_half

        def s1_desc(c):
            return pltpu.make_async_remote_copy(
                src_ref=x_ref.at[pl.ds(my_half_src + c * ck, ck), :],
                dst_ref=out_ref.at[pl.ds(my_y * m_per + my_half_src + c * ck, ck), :],
                send_sem=s1_send.at[c],
                recv_sem=s1_recv.at[c],
                device_id=y_peer,
                device_id_type=pl.DeviceIdType.MESH,
            )

        def s1_recv_desc(c):
            return pltpu.make_async_remote_copy(
                src_ref=x_ref.at[pl.ds(my_half_src + c * ck, ck), :],
                dst_ref=out_ref.at[pl.ds(s1_rows + c * ck, ck), :],
                send_sem=s1_send.at[c],
                recv_sem=s1_recv.at[c],
                device_id=y_peer,
                device_id_type=pl.DeviceIdType.MESH,
            )

        def s2_desc(c):
            return pltpu.make_async_remote_copy(
                src_ref=out_ref.at[pl.ds(s1_rows + c * ck, ck), :],
                dst_ref=out_ref.at[pl.ds(s1_rows + c * ck, ck), :],
                send_sem=s2_send.at[c],
                recv_sem=s2_recv.at[c],
                device_id=x_peer,
                device_id_type=pl.DeviceIdType.MESH,
            )

        def s2_recv_desc(c):
            return pltpu.make_async_remote_copy(
                src_ref=out_ref.at[pl.ds(s2_rows + c * ck, ck), :],
                dst_ref=out_ref.at[pl.ds(s2_rows + c * ck, ck), :],
                send_sem=s2_send.at[c],
                recv_sem=s2_recv.at[c],
                device_id=x_peer,
                device_id_type=pl.DeviceIdType.MESH,
            )

        def h2v(c):
            return pltpu.make_async_copy(
                x_ref.at[pl.ds(c * ckv, ckv), :],
                vbuf.at[c % N_SLOTS],
                h2v_sem.at[c % N_SLOTS],
            )

        def v2h(c):
            return pltpu.make_async_copy(
                vbuf.at[c % N_SLOTS],
                out_ref.at[pl.ds(my_y * m_per + c * ckv, ckv), :],
                v2h_sem.at[c % N_SLOTS],
            )

        for c in range(N_CHUNKS):
            s1_desc(c).start()

        for c in range(N_CHUNKS):
            if c >= N_SLOTS:
                v2h(c - N_SLOTS).wait()
            h2v(c).start()
            h2v(c).wait()
            v2h(c).start()
        for c in range(N_CHUNKS - N_SLOTS, N_CHUNKS):
            v2h(c).wait()

        for c in range(N_CHUNKS):
            s1_recv_desc(c).wait_recv()
            s2_desc(c).start()

        for c in range(N_CHUNKS):
            s2_recv_desc(c).wait_recv()
        for c in range(N_CHUNKS):
            s1_desc(c).wait_send()
            s2_desc(c).wait_send()

    return pl.pallas_call(
        body,
        out_shape=jax.ShapeDtypeStruct((m_glob, n), x.dtype),
        in_specs=[pl.BlockSpec(memory_space=pl.ANY)],
        out_specs=pl.BlockSpec(memory_space=pl.ANY),
        scratch_shapes=[
            pltpu.SemaphoreType.DMA((N_CHUNKS,)),
            pltpu.SemaphoreType.DMA((N_CHUNKS,)),
            pltpu.SemaphoreType.DMA((N_CHUNKS,)),
            pltpu.SemaphoreType.DMA((N_CHUNKS,)),
            pltpu.VMEM((N_SLOTS, m_per // N_CHUNKS, n), x.dtype),
            pltpu.SemaphoreType.DMA((N_SLOTS,)),
            pltpu.SemaphoreType.DMA((N_SLOTS,)),
        ],
        compiler_params=pltpu.CompilerParams(collective_id=0),
    )(x)


# device time: 920114 ns/iter; 1.0900x vs baseline; 1.0900x over previous
import jax
import jax.numpy as jnp
from jax import lax
from jax.experimental import pallas as pl
from jax.experimental.pallas import tpu as pltpu

N_CHUNKS = 32
N_SLOTS = 4


def kernel(x):
    m_per, n = x.shape
    m_glob = 2 * m_per
    m_half = m_per // 2
    ck = m_half // N_CHUNKS
    ckv = m_per // N_CHUNKS

    def body(x_ref, out_ref, s1_send, s1_recv, s2_send, s2_recv,
             vbuf, h2v_sem, v2h_sem):
        my_x = lax.axis_index("x")
        my_y = lax.axis_index("y")
        y_peer = (my_x, 1 - my_y)
        x_peer = (1 - my_x, my_y)

        barrier_sem = pltpu.get_barrier_semaphore()
        for peer in (y_peer, x_peer):
            pl.semaphore_signal(
                barrier_sem, inc=1, device_id=peer,
                device_id_type=pl.DeviceIdType.MESH,
            )
        pl.semaphore_wait(barrier_sem, 2)

        my_half_src = my_x * m_half
        s1_rows = (1 - my_y) * m_per + my_x * m_half
        s2_rows = (1 - my_y) * m_per + (1 - my_x) * m_half

        def s1_desc(c):
            return pltpu.make_async_remote_copy(
                src_ref=x_ref.at[pl.ds(my_half_src + c * ck, ck), :],
                dst_ref=out_ref.at[pl.ds(my_y * m_per + my_half_src + c * ck, ck), :],
                send_sem=s1_send.at[c],
                recv_sem=s1_recv.at[c],
                device_id=y_peer,
                device_id_type=pl.DeviceIdType.MESH,
            )

        def s1_recv_desc(c):
            return pltpu.make_async_remote_copy(
                src_ref=x_ref.at[pl.ds(my_half_src + c * ck, ck), :],
                dst_ref=out_ref.at[pl.ds(s1_rows + c * ck, ck), :],
                send_sem=s1_send.at[c],
                recv_sem=s1_recv.at[c],
                device_id=y_peer,
                device_id_type=pl.DeviceIdType.MESH,
            )

        def s2_desc(c):
            return pltpu.make_async_remote_copy(
                src_ref=out_ref.at[pl.ds(s1_rows + c * ck, ck), :],
                dst_ref=out_ref.at[pl.ds(s1_rows + c * ck, ck), :],
                send_sem=s2_send.at[c],
                recv_sem=s2_recv.at[c],
                device_id=x_peer,
                device_id_type=pl.DeviceIdType.MESH,
            )

        def s2_recv_desc(c):
            return pltpu.make_async_remote_copy(
                src_ref=out_ref.at[pl.ds(s2_rows + c * ck, ck), :],
                dst_ref=out_ref.at[pl.ds(s2_rows + c * ck, ck), :],
                send_sem=s2_send.at[c],
                recv_sem=s2_recv.at[c],
                device_id=x_peer,
                device_id_type=pl.DeviceIdType.MESH,
            )

        def h2v(c):
            return pltpu.make_async_copy(
                x_ref.at[pl.ds(c * ckv, ckv), :],
                vbuf.at[c % N_SLOTS],
                h2v_sem.at[c % N_SLOTS],
            )

        def v2h(c):
            return pltpu.make_async_copy(
                vbuf.at[c % N_SLOTS],
                out_ref.at[pl.ds(my_y * m_per + c * ckv, ckv), :],
                v2h_sem.at[c % N_SLOTS],
            )

        for c in range(N_CHUNKS):
            s1_desc(c).start()

        for c in range(N_CHUNKS):
            s1_recv_desc(c).wait_recv()
            s2_desc(c).start()
            if c >= N_SLOTS:
                v2h(c - N_SLOTS).wait()
            h2v(c).start()
            h2v(c).wait()
            v2h(c).start()

        for c in range(N_CHUNKS - N_SLOTS, N_CHUNKS):
            v2h(c).wait()
        for c in range(N_CHUNKS):
            s2_recv_desc(c).wait_recv()
        for c in range(N_CHUNKS):
            s1_desc(c).wait_send()
            s2_desc(c).wait_send()

    return pl.pallas_call(
        body,
        out_shape=jax.ShapeDtypeStruct((m_glob, n), x.dtype),
        in_specs=[pl.BlockSpec(memory_space=pl.ANY)],
        out_specs=pl.BlockSpec(memory_space=pl.ANY),
        scratch_shapes=[
            pltpu.SemaphoreType.DMA((N_CHUNKS,)),
            pltpu.SemaphoreType.DMA((N_CHUNKS,)),
            pltpu.SemaphoreType.DMA((N_CHUNKS,)),
            pltpu.SemaphoreType.DMA((N_CHUNKS,)),
            pltpu.VMEM((N_SLOTS, m_per // N_CHUNKS, n), x.dtype),
            pltpu.SemaphoreType.DMA((N_SLOTS,)),
            pltpu.SemaphoreType.DMA((N_SLOTS,)),
        ],
        compiler_params=pltpu.CompilerParams(collective_id=0),
    )(x)


# device time: 909772 ns/iter; 1.1024x vs baseline; 1.0114x over previous
import jax
import jax.numpy as jnp
from jax import lax
from jax.experimental import pallas as pl
from jax.experimental.pallas import tpu as pltpu

N_CHUNKS = 64
N_SLOTS = 4


def kernel(x):
    m_per, n = x.shape
    m_glob = 2 * m_per
    m_half = m_per // 2
    ck = m_half // N_CHUNKS
    ckv = m_per // N_CHUNKS

    def body(x_ref, out_ref, s1_send, s1_recv, s2_send, s2_recv,
             vbuf, h2v_sem, v2h_sem):
        my_x = lax.axis_index("x")
        my_y = lax.axis_index("y")
        y_peer = (my_x, 1 - my_y)
        x_peer = (1 - my_x, my_y)

        barrier_sem = pltpu.get_barrier_semaphore()
        for peer in (y_peer, x_peer):
            pl.semaphore_signal(
                barrier_sem, inc=1, device_id=peer,
                device_id_type=pl.DeviceIdType.MESH,
            )
        pl.semaphore_wait(barrier_sem, 2)

        my_half_src = my_x * m_half
        s1_rows = (1 - my_y) * m_per + my_x * m_half
        s2_rows = (1 - my_y) * m_per + (1 - my_x) * m_half

        def s1_desc(c):
            return pltpu.make_async_remote_copy(
                src_ref=x_ref.at[pl.ds(my_half_src + c * ck, ck), :],
                dst_ref=out_ref.at[pl.ds(my_y * m_per + my_half_src + c * ck, ck), :],
                send_sem=s1_send.at[c],
                recv_sem=s1_recv.at[c],
                device_id=y_peer,
                device_id_type=pl.DeviceIdType.MESH,
            )

        def s1_recv_desc(c):
            return pltpu.make_async_remote_copy(
                src_ref=x_ref.at[pl.ds(my_half_src + c * ck, ck), :],
                dst_ref=out_ref.at[pl.ds(s1_rows + c * ck, ck), :],
                send_sem=s1_send.at[c],
                recv_sem=s1_recv.at[c],
                device_id=y_peer,
                device_id_type=pl.DeviceIdType.MESH,
            )

        def s2_desc(c):
            return pltpu.make_async_remote_copy(
                src_ref=out_ref.at[pl.ds(s1_rows + c * ck, ck), :],
                dst_ref=out_ref.at[pl.ds(s1_rows + c * ck, ck), :],
                send_sem=s2_send.at[c],
                recv_sem=s2_recv.at[c],
                device_id=x_peer,
                device_id_type=pl.DeviceIdType.MESH,
            )

        def s2_recv_desc(c):
            return pltpu.make_async_remote_copy(
                src_ref=out_ref.at[pl.ds(s2_rows + c * ck, ck), :],
                dst_ref=out_ref.at[pl.ds(s2_rows + c * ck, ck), :],
                send_sem=s2_send.at[c],
                recv_sem=s2_recv.at[c],
                device_id=x_peer,
                device_id_type=pl.DeviceIdType.MESH,
            )

        def h2v(c):
            return pltpu.make_async_copy(
                x_ref.at[pl.ds(c * ckv, ckv), :],
                vbuf.at[c % N_SLOTS],
                h2v_sem.at[c % N_SLOTS],
            )

        def v2h(c):
            return pltpu.make_async_copy(
                vbuf.at[c % N_SLOTS],
                out_ref.at[pl.ds(my_y * m_per + c * ckv, ckv), :],
                v2h_sem.at[c % N_SLOTS],
            )

        for c in range(N_CHUNKS):
            s1_desc(c).start()

        for c in range(N_CHUNKS):
            s1_recv_desc(c).wait_recv()
            s2_desc(c).start()
            if c >= N_SLOTS:
                v2h(c - N_SLOTS).wait()
            h2v(c).start()
            h2v(c).wait()
            v2h(c).start()

        for c in range(N_CHUNKS - N_SLOTS, N_CHUNKS):
            v2h(c).wait()
        for c in range(N_CHUNKS):
            s2_recv_desc(c).wait_recv()
        for c in range(N_CHUNKS):
            s1_desc(c).wait_send()
            s2_desc(c).wait_send()

    return pl.pallas_call(
        body,
        out_shape=jax.ShapeDtypeStruct((m_glob, n), x.dtype),
        in_specs=[pl.BlockSpec(memory_space=pl.ANY)],
        out_specs=pl.BlockSpec(memory_space=pl.ANY),
        scratch_shapes=[
            pltpu.SemaphoreType.DMA((N_CHUNKS,)),
            pltpu.SemaphoreType.DMA((N_CHUNKS,)),
            pltpu.SemaphoreType.DMA((N_CHUNKS,)),
            pltpu.SemaphoreType.DMA((N_CHUNKS,)),
            pltpu.VMEM((N_SLOTS, m_per // N_CHUNKS, n), x.dtype),
            pltpu.SemaphoreType.DMA((N_SLOTS,)),
            pltpu.SemaphoreType.DMA((N_SLOTS,)),
        ],
        compiler_params=pltpu.CompilerParams(collective_id=0),
    )(x)
